# baseline (speedup 1.0000x reference)
import functools

import jax
import jax.numpy as jnp
from jax import lax
from jax.experimental import pallas as pl
from jax.experimental.pallas import tpu as pltpu
from jax.experimental.pallas import tpu_sc as plsc

_NC = 2
_NS = 16
_NW = _NC * _NS
_K = 64
_CHUNK = 128


def _sc_call(B):
    b_per_w = B // _NW
    n_chunks = b_per_w // _CHUNK
    mesh = plsc.VectorSubcoreMesh(core_axis_name="c", subcore_axis_name="s")

    @functools.partial(
        pl.kernel,
        mesh=mesh,
        compiler_params=pltpu.CompilerParams(needs_layout_passes=False,
                                              use_tc_tiling_on_sc=False),
        out_type=jax.ShapeDtypeStruct((B,), jnp.float32),
        scratch_types=[
            pltpu.VMEM((b_per_w,), jnp.int32),
            pltpu.VMEM((b_per_w,), jnp.int32),
            pltpu.VMEM((b_per_w,), jnp.int32),
            pltpu.VMEM((b_per_w,), jnp.int32),
            pltpu.VMEM((_CHUNK, _K), jnp.float32),
            pltpu.VMEM((_CHUNK, _K), jnp.float32),
            pltpu.VMEM((_CHUNK, _K), jnp.float32),
            pltpu.VMEM((_CHUNK, _K), jnp.float32),
            pltpu.VMEM((_CHUNK, _K), jnp.float32),
            pltpu.VMEM((_CHUNK, _K), jnp.float32),
            pltpu.VMEM((b_per_w,), jnp.float32),
            pltpu.SemaphoreType.DMA,
        ],
    )
    def run(u_hbm, i_hbm, p_hbm, n_hbm, uv_hbm, iv_hbm, tu_hbm, ti_hbm,
            out_hbm, u_v, i_v, p_v, n_v, ur, ir, tup, tip, tun, tin,
            out_v, sem):
        wid = lax.axis_index("s") * _NC + lax.axis_index("c")
        base = wid * b_per_w
        pltpu.sync_copy(u_hbm.at[pl.ds(base, b_per_w)], u_v)
        pltpu.sync_copy(i_hbm.at[pl.ds(base, b_per_w)], i_v)
        pltpu.sync_copy(p_hbm.at[pl.ds(base, b_per_w)], p_v)
        pltpu.sync_copy(n_hbm.at[pl.ds(base, b_per_w)], n_v)

        def chunk_body(c, _):
            off = c * _CHUNK
            cps = [
                pltpu.async_copy(uv_hbm.at[u_v.at[pl.ds(off, _CHUNK)]], ur, sem),
                pltpu.async_copy(iv_hbm.at[i_v.at[pl.ds(off, _CHUNK)]], ir, sem),
                pltpu.async_copy(tu_hbm.at[p_v.at[pl.ds(off, _CHUNK)]], tup, sem),
                pltpu.async_copy(ti_hbm.at[p_v.at[pl.ds(off, _CHUNK)]], tip, sem),
                pltpu.async_copy(tu_hbm.at[n_v.at[pl.ds(off, _CHUNK)]], tun, sem),
                pltpu.async_copy(ti_hbm.at[n_v.at[pl.ds(off, _CHUNK)]], tin, sem),
            ]
            for cp in cps:
                cp.wait()

            def group_body(g, _):
                rows16 = g * 16 + lax.iota(jnp.int32, 16)
                acc = jnp.zeros((16,), jnp.float32)
                for k in range(_K):
                    kv = jnp.full((16,), k, jnp.int32)
                    acc = acc + plsc.load_gather(ur, [rows16, kv]) * (
                        plsc.load_gather(tup, [rows16, kv])
                        - plsc.load_gather(tun, [rows16, kv]))
                    acc = acc + plsc.load_gather(ir, [rows16, kv]) * (
                        plsc.load_gather(tip, [rows16, kv])
                        - plsc.load_gather(tin, [rows16, kv]))
                out_v[pl.ds(off + g * 16, 16)] = acc
                return 0

            lax.fori_loop(0, _CHUNK // 16, group_body, 0)
            return 0

        lax.fori_loop(0, n_chunks, chunk_body, 0)
        pltpu.sync_copy(out_v, out_hbm.at[pl.ds(base, b_per_w)])

    return run


def kernel(x, userVecs, itemVecs, tagUserVecs, tagItemVecs):
    if x.ndim == 1:
        x = x.reshape(1, x.shape[0])
    B = x.shape[0]
    xc = x.T
    u_id = xc[0]
    i_id = xc[1]
    p_id = xc[2]
    n_id = xc[3]
    return _sc_call(B)(u_id, i_id, p_id, n_id, userVecs, itemVecs,
                       tagUserVecs, tagItemVecs)
